# baseline (speedup 1.0000x reference)
import functools

import jax
import jax.numpy as jnp
from jax import lax
from jax.experimental import pallas as pl
from jax.experimental.pallas import tpu as pltpu
from jax.experimental.pallas import tpu_sc as plsc

N_NODES = 10000
N_EDGES = 320000
D_FEAT = 128
D_EDGE = 16
D_U = 128
N_GRAPHS = 16
OUT_DIM = 128

NC = 2
NS = 16
NW = NC * NS
E_PER_TILE = N_EDGES // NW
CHUNK = 40
N_CHUNKS = E_PER_TILE // CHUNK
ROW_STRIPE = 1000
N_STRIPES = N_NODES // ROW_STRIPE

NB = 1000
EB = 2000


def _node_pre_body(x_ref, batch_ref, u_ref, w_sd_ref, w_u_ref, a_ref, b_ref):
    xb = x_ref[...]
    ab = jnp.dot(xb, w_sd_ref[...], preferred_element_type=jnp.float32)
    u_proj = jnp.dot(u_ref[...], w_u_ref[...], preferred_element_type=jnp.float32)
    oh = (batch_ref[...] == lax.broadcasted_iota(jnp.int32, (NB, N_GRAPHS), 1)
          ).astype(jnp.float32)
    a_ref[...] = ab[:, :OUT_DIM] + jnp.dot(
        oh, u_proj, preferred_element_type=jnp.float32)
    b_ref[...] = ab[:, OUT_DIM:]


def _edge_pre_body(ea_ref, w_e_ref, b1_ref, pe_ref):
    pe_ref[...] = jnp.dot(ea_ref[...], w_e_ref[...],
                          preferred_element_type=jnp.float32) + b1_ref[...]


def _final_body(x_ref, agg_ref, batch_ref, u_ref, w2_ref, b2_ref, o_ref):
    w2 = w2_ref[...]
    agg = agg_ref[0] + agg_ref[1]
    acc = jnp.dot(x_ref[...], w2[:D_FEAT], preferred_element_type=jnp.float32)
    acc += jnp.dot(agg, w2[D_FEAT:D_FEAT + OUT_DIM],
                   preferred_element_type=jnp.float32)
    u_proj = jnp.dot(u_ref[...], w2[D_FEAT + OUT_DIM:],
                     preferred_element_type=jnp.float32)
    oh = (batch_ref[...] == lax.broadcasted_iota(jnp.int32, (NB, N_GRAPHS), 1)
          ).astype(jnp.float32)
    acc += jnp.dot(oh, u_proj, preferred_element_type=jnp.float32)
    o_ref[...] = jnp.maximum(acc + b2_ref[...], 0.0)


def _sc_edge_body(a_hbm, b_hbm, pe_hbm, src_hbm, dst_hbm, zero_hbm, out_hbm,
                  si0, di0, si1, di1, a0, b0, p0, a1, b1_, p1, buf_e, agg,
                  sem0, sem1):
    c = lax.axis_index("c")
    s = lax.axis_index("s")
    wid = c * NS + s

    @pl.when(s < N_STRIPES)
    def _zero():
        pltpu.sync_copy(zero_hbm.at[pl.ds(s * ROW_STRIPE, ROW_STRIPE)],
                        agg.at[pl.ds(s * ROW_STRIPE, ROW_STRIPE)])

    plsc.subcore_barrier()

    def load_idx(t, si, di):
        base = wid * E_PER_TILE + t * CHUNK
        pltpu.sync_copy(src_hbm.at[pl.ds(base, CHUNK)], si)
        pltpu.sync_copy(dst_hbm.at[pl.ds(base, CHUNK)], di)

    def issue_g(t, si, di, ba, bb, bp, sem):
        cps = (pltpu.async_copy(a_hbm.at[si], ba, sem),
               pltpu.async_copy(b_hbm.at[di], bb, sem),
               pltpu.async_copy(
                   pe_hbm.at[pl.ds(wid * E_PER_TILE + t * CHUNK, CHUNK)],
                   bp, sem))
        return cps

    def drain(cps):
        for cp in cps:
            cp.wait()

    def compute(ba, bb, bp):
        def row_body(r, rc):
            for j in range(OUT_DIM // 16):
                sl = pl.ds(j * 16, 16)
                v = ba[r, sl] + bb[r, sl] + bp[r, sl]
                buf_e[r, sl] = jnp.maximum(v, 0.0)
            return rc

        lax.fori_loop(0, CHUNK, row_body, 0)

    def scatter(di, bp):
        pltpu.sync_copy(buf_e, agg.at[di], add=True)

    load_idx(0, si0, di0)
    drain(issue_g(0, si0, di0, a0, b0, p0, sem0))

    def pair_body(i, carry):
        t0 = 2 * i
        load_idx(t0 + 1, si1, di1)
        cps1 = issue_g(t0 + 1, si1, di1, a1, b1_, p1, sem1)
        compute(a0, b0, p0)
        drain(cps1)
        scatter(di0, p0)
        load_idx(t0 + 2, si0, di0)
        cps0 = issue_g(t0 + 2, si0, di0, a0, b0, p0, sem0)
        compute(a1, b1_, p1)
        drain(cps0)
        scatter(di1, p1)
        return carry

    lax.fori_loop(0, N_CHUNKS // 2 - 1, pair_body, 0)
    load_idx(N_CHUNKS - 1, si1, di1)
    cps1 = issue_g(N_CHUNKS - 1, si1, di1, a1, b1_, p1, sem1)
    compute(a0, b0, p0)
    drain(cps1)
    scatter(di0, p0)
    compute(a1, b1_, p1)
    scatter(di1, p1)

    plsc.subcore_barrier()

    @pl.when(s < N_STRIPES)
    def _writeback():
        pltpu.sync_copy(agg.at[pl.ds(s * ROW_STRIPE, ROW_STRIPE)],
                        out_hbm.at[c, pl.ds(s * ROW_STRIPE, ROW_STRIPE)])


_sc_edge_kernel = functools.partial(
    pl.kernel,
    out_type=jax.ShapeDtypeStruct((NC, N_NODES, OUT_DIM), jnp.float32),
    mesh=plsc.VectorSubcoreMesh(core_axis_name="c", subcore_axis_name="s",
                                num_cores=NC, num_subcores=NS),
    scratch_types=[
        pltpu.VMEM((CHUNK,), jnp.int32),
        pltpu.VMEM((CHUNK,), jnp.int32),
        pltpu.VMEM((CHUNK,), jnp.int32),
        pltpu.VMEM((CHUNK,), jnp.int32),
        pltpu.VMEM((CHUNK, OUT_DIM), jnp.float32),
        pltpu.VMEM((CHUNK, OUT_DIM), jnp.float32),
        pltpu.VMEM((CHUNK, OUT_DIM), jnp.float32),
        pltpu.VMEM((CHUNK, OUT_DIM), jnp.float32),
        pltpu.VMEM((CHUNK, OUT_DIM), jnp.float32),
        pltpu.VMEM((CHUNK, OUT_DIM), jnp.float32),
        pltpu.VMEM((CHUNK, OUT_DIM), jnp.float32),
        pltpu.VMEM_SHARED((N_NODES, OUT_DIM), jnp.float32),
        pltpu.SemaphoreType.DMA,
        pltpu.SemaphoreType.DMA,
    ],
)(_sc_edge_body)


def kernel(x, edge_index, edge_attr, u, batch, W1, b1, W2, b2):
    x = x.astype(jnp.float32)
    src = edge_index[0].astype(jnp.int32)
    dst = edge_index[1].astype(jnp.int32)
    batch2d = batch.astype(jnp.int32).reshape(N_NODES, 1)

    w_sd = jnp.concatenate([W1[:D_FEAT], W1[D_FEAT:2 * D_FEAT]], axis=1)
    w_e = W1[2 * D_FEAT:2 * D_FEAT + D_EDGE]
    w_u = W1[2 * D_FEAT + D_EDGE:]
    b1r = b1.reshape(1, OUT_DIM)
    b2r = b2.reshape(1, OUT_DIM)

    n_grid = N_NODES // NB
    a_tab, b_tab = pl.pallas_call(
        _node_pre_body,
        grid=(n_grid,),
        in_specs=[
            pl.BlockSpec((NB, D_FEAT), lambda i: (i, 0)),
            pl.BlockSpec((NB, 1), lambda i: (i, 0)),
            pl.BlockSpec((N_GRAPHS, D_U), lambda i: (0, 0)),
            pl.BlockSpec((D_FEAT, 2 * OUT_DIM), lambda i: (0, 0)),
            pl.BlockSpec((D_U, OUT_DIM), lambda i: (0, 0)),
        ],
        out_specs=[
            pl.BlockSpec((NB, OUT_DIM), lambda i: (i, 0)),
            pl.BlockSpec((NB, OUT_DIM), lambda i: (i, 0)),
        ],
        out_shape=[
            jax.ShapeDtypeStruct((N_NODES, OUT_DIM), jnp.float32),
            jax.ShapeDtypeStruct((N_NODES, OUT_DIM), jnp.float32),
        ],
    )(x, batch2d, u, w_sd, w_u)

    pe = pl.pallas_call(
        _edge_pre_body,
        grid=(N_EDGES // EB,),
        in_specs=[
            pl.BlockSpec((EB, D_EDGE), lambda i: (i, 0)),
            pl.BlockSpec((D_EDGE, OUT_DIM), lambda i: (0, 0)),
            pl.BlockSpec((1, OUT_DIM), lambda i: (0, 0)),
        ],
        out_specs=pl.BlockSpec((EB, OUT_DIM), lambda i: (i, 0)),
        out_shape=jax.ShapeDtypeStruct((N_EDGES, OUT_DIM), jnp.float32),
    )(edge_attr, w_e, b1r)

    zeros = jnp.zeros((N_NODES, OUT_DIM), jnp.float32)
    agg2 = _sc_edge_kernel(a_tab, b_tab, pe, src, dst, zeros)

    out = pl.pallas_call(
        _final_body,
        grid=(n_grid,),
        in_specs=[
            pl.BlockSpec((NB, D_FEAT), lambda i: (i, 0)),
            pl.BlockSpec((NC, NB, OUT_DIM), lambda i: (0, i, 0)),
            pl.BlockSpec((NB, 1), lambda i: (i, 0)),
            pl.BlockSpec((N_GRAPHS, D_U), lambda i: (0, 0)),
            pl.BlockSpec((D_FEAT + OUT_DIM + D_U, OUT_DIM), lambda i: (0, 0)),
            pl.BlockSpec((1, OUT_DIM), lambda i: (0, 0)),
        ],
        out_specs=pl.BlockSpec((NB, OUT_DIM), lambda i: (i, 0)),
        out_shape=jax.ShapeDtypeStruct((N_NODES, OUT_DIM), jnp.float32),
    )(x, agg2, batch2d, u, W2, b2r)
    return out


# speedup vs baseline: 1.2243x; 1.2243x over previous
import functools

import jax
import jax.numpy as jnp
from jax import lax
from jax.experimental import pallas as pl
from jax.experimental.pallas import tpu as pltpu
from jax.experimental.pallas import tpu_sc as plsc

N_NODES = 10000
N_EDGES = 320000
D_FEAT = 128
D_EDGE = 16
D_U = 128
N_GRAPHS = 16
OUT_DIM = 128

NC = 2
NS = 16
NW = NC * NS
E_PER_TILE = N_EDGES // NW
CHUNK = 40
N_CHUNKS = E_PER_TILE // CHUNK
ROW_STRIPE = 1000
N_STRIPES = N_NODES // ROW_STRIPE

NB = 1000
EB = 2000


def _node_pre_body(x_ref, batch_ref, u_ref, w_sd_ref, w_u_ref, a_ref, b_ref):
    xb = x_ref[...]
    ab = jnp.dot(xb, w_sd_ref[...], preferred_element_type=jnp.float32)
    u_proj = jnp.dot(u_ref[...], w_u_ref[...], preferred_element_type=jnp.float32)
    oh = (batch_ref[...] == lax.broadcasted_iota(jnp.int32, (NB, N_GRAPHS), 1)
          ).astype(jnp.float32)
    a_ref[...] = ab[:, :OUT_DIM] + jnp.dot(
        oh, u_proj, preferred_element_type=jnp.float32)
    b_ref[...] = ab[:, OUT_DIM:]


def _edge_pre_body(ea_ref, w_e_ref, b1_ref, pe_ref):
    pe_ref[...] = jnp.dot(ea_ref[...], w_e_ref[...],
                          preferred_element_type=jnp.float32) + b1_ref[...]


def _final_body(x_ref, agg_ref, batch_ref, u_ref, w2_ref, b2_ref, o_ref):
    w2 = w2_ref[...]
    agg = agg_ref[0] + agg_ref[1]
    acc = jnp.dot(x_ref[...], w2[:D_FEAT], preferred_element_type=jnp.float32)
    acc += jnp.dot(agg, w2[D_FEAT:D_FEAT + OUT_DIM],
                   preferred_element_type=jnp.float32)
    u_proj = jnp.dot(u_ref[...], w2[D_FEAT + OUT_DIM:],
                     preferred_element_type=jnp.float32)
    oh = (batch_ref[...] == lax.broadcasted_iota(jnp.int32, (NB, N_GRAPHS), 1)
          ).astype(jnp.float32)
    acc += jnp.dot(oh, u_proj, preferred_element_type=jnp.float32)
    o_ref[...] = jnp.maximum(acc + b2_ref[...], 0.0)


def _sc_edge_body(a_hbm, b_hbm, pe_hbm, src_hbm, dst_hbm, zero_hbm, out_hbm,
                  si0, di0, si1, di1, a0, b0, p0, a1, b1_, p1, agg,
                  sem0, sem1):
    c = lax.axis_index("c")
    s = lax.axis_index("s")
    wid = c * NS + s

    @pl.when(s < N_STRIPES)
    def _zero():
        pltpu.sync_copy(zero_hbm.at[pl.ds(s * ROW_STRIPE, ROW_STRIPE)],
                        agg.at[pl.ds(s * ROW_STRIPE, ROW_STRIPE)])

    plsc.subcore_barrier()

    def load_idx(t, si, di):
        base = wid * E_PER_TILE + t * CHUNK
        pltpu.sync_copy(src_hbm.at[pl.ds(base, CHUNK)], si)
        pltpu.sync_copy(dst_hbm.at[pl.ds(base, CHUNK)], di)

    def issue_g(t, si, di, ba, bb, bp, sem):
        pltpu.async_copy(a_hbm.at[si], ba, sem)
        pltpu.async_copy(b_hbm.at[di], bb, sem)
        pltpu.async_copy(
            pe_hbm.at[pl.ds(wid * E_PER_TILE + t * CHUNK, CHUNK)], bp, sem)

    def drain(ba, bb, bp, sem):
        pltpu.make_async_copy(pe_hbm.at[pl.ds(0, CHUNK)], ba, sem).wait()
        pltpu.make_async_copy(pe_hbm.at[pl.ds(0, CHUNK)], bb, sem).wait()
        pltpu.make_async_copy(pe_hbm.at[pl.ds(0, CHUNK)], bp, sem).wait()

    def compute_scatter(di, ba, bb, bp):
        def row_body(r, rc):
            for j in range(OUT_DIM // 16):
                sl = pl.ds(j * 16, 16)
                v = ba[r, sl] + bb[r, sl] + bp[r, sl]
                bp[r, sl] = jnp.maximum(v, 0.0)
            return rc

        lax.fori_loop(0, CHUNK, row_body, 0)
        pltpu.sync_copy(bp, agg.at[di], add=True)

    load_idx(0, si0, di0)
    issue_g(0, si0, di0, a0, b0, p0, sem0)
    load_idx(1, si1, di1)

    def pair_body(i, carry):
        t0 = 2 * i
        issue_g(t0 + 1, si1, di1, a1, b1_, p1, sem1)
        drain(a0, b0, p0, sem0)
        compute_scatter(di0, a0, b0, p0)
        load_idx(t0 + 2, si0, di0)
        issue_g(t0 + 2, si0, di0, a0, b0, p0, sem0)
        drain(a1, b1_, p1, sem1)
        compute_scatter(di1, a1, b1_, p1)
        load_idx(t0 + 3, si1, di1)
        return carry

    lax.fori_loop(0, N_CHUNKS // 2 - 1, pair_body, 0)
    issue_g(N_CHUNKS - 1, si1, di1, a1, b1_, p1, sem1)
    drain(a0, b0, p0, sem0)
    compute_scatter(di0, a0, b0, p0)
    drain(a1, b1_, p1, sem1)
    compute_scatter(di1, a1, b1_, p1)

    plsc.subcore_barrier()

    @pl.when(s < N_STRIPES)
    def _writeback():
        pltpu.sync_copy(agg.at[pl.ds(s * ROW_STRIPE, ROW_STRIPE)],
                        out_hbm.at[c, pl.ds(s * ROW_STRIPE, ROW_STRIPE)])


_sc_edge_kernel = functools.partial(
    pl.kernel,
    out_type=jax.ShapeDtypeStruct((NC, N_NODES, OUT_DIM), jnp.float32),
    mesh=plsc.VectorSubcoreMesh(core_axis_name="c", subcore_axis_name="s",
                                num_cores=NC, num_subcores=NS),
    scratch_types=[
        pltpu.VMEM((CHUNK,), jnp.int32),
        pltpu.VMEM((CHUNK,), jnp.int32),
        pltpu.VMEM((CHUNK,), jnp.int32),
        pltpu.VMEM((CHUNK,), jnp.int32),
        pltpu.VMEM((CHUNK, OUT_DIM), jnp.float32),
        pltpu.VMEM((CHUNK, OUT_DIM), jnp.float32),
        pltpu.VMEM((CHUNK, OUT_DIM), jnp.float32),
        pltpu.VMEM((CHUNK, OUT_DIM), jnp.float32),
        pltpu.VMEM((CHUNK, OUT_DIM), jnp.float32),
        pltpu.VMEM((CHUNK, OUT_DIM), jnp.float32),
        pltpu.VMEM_SHARED((N_NODES, OUT_DIM), jnp.float32),
        pltpu.SemaphoreType.DMA,
        pltpu.SemaphoreType.DMA,
    ],
)(_sc_edge_body)


def kernel(x, edge_index, edge_attr, u, batch, W1, b1, W2, b2):
    x = x.astype(jnp.float32)
    src = edge_index[0].astype(jnp.int32)
    dst = edge_index[1].astype(jnp.int32)
    batch2d = batch.astype(jnp.int32).reshape(N_NODES, 1)

    w_sd = jnp.concatenate([W1[:D_FEAT], W1[D_FEAT:2 * D_FEAT]], axis=1)
    w_e = W1[2 * D_FEAT:2 * D_FEAT + D_EDGE]
    w_u = W1[2 * D_FEAT + D_EDGE:]
    b1r = b1.reshape(1, OUT_DIM)
    b2r = b2.reshape(1, OUT_DIM)

    n_grid = N_NODES // NB
    a_tab, b_tab = pl.pallas_call(
        _node_pre_body,
        grid=(n_grid,),
        in_specs=[
            pl.BlockSpec((NB, D_FEAT), lambda i: (i, 0)),
            pl.BlockSpec((NB, 1), lambda i: (i, 0)),
            pl.BlockSpec((N_GRAPHS, D_U), lambda i: (0, 0)),
            pl.BlockSpec((D_FEAT, 2 * OUT_DIM), lambda i: (0, 0)),
            pl.BlockSpec((D_U, OUT_DIM), lambda i: (0, 0)),
        ],
        out_specs=[
            pl.BlockSpec((NB, OUT_DIM), lambda i: (i, 0)),
            pl.BlockSpec((NB, OUT_DIM), lambda i: (i, 0)),
        ],
        out_shape=[
            jax.ShapeDtypeStruct((N_NODES, OUT_DIM), jnp.float32),
            jax.ShapeDtypeStruct((N_NODES, OUT_DIM), jnp.float32),
        ],
    )(x, batch2d, u, w_sd, w_u)

    pe = pl.pallas_call(
        _edge_pre_body,
        grid=(N_EDGES // EB,),
        in_specs=[
            pl.BlockSpec((EB, D_EDGE), lambda i: (i, 0)),
            pl.BlockSpec((D_EDGE, OUT_DIM), lambda i: (0, 0)),
            pl.BlockSpec((1, OUT_DIM), lambda i: (0, 0)),
        ],
        out_specs=pl.BlockSpec((EB, OUT_DIM), lambda i: (i, 0)),
        out_shape=jax.ShapeDtypeStruct((N_EDGES, OUT_DIM), jnp.float32),
    )(edge_attr, w_e, b1r)

    zeros = jnp.zeros((N_NODES, OUT_DIM), jnp.float32)
    agg2 = _sc_edge_kernel(a_tab, b_tab, pe, src, dst, zeros)

    out = pl.pallas_call(
        _final_body,
        grid=(n_grid,),
        in_specs=[
            pl.BlockSpec((NB, D_FEAT), lambda i: (i, 0)),
            pl.BlockSpec((NC, NB, OUT_DIM), lambda i: (0, i, 0)),
            pl.BlockSpec((NB, 1), lambda i: (i, 0)),
            pl.BlockSpec((N_GRAPHS, D_U), lambda i: (0, 0)),
            pl.BlockSpec((D_FEAT + OUT_DIM + D_U, OUT_DIM), lambda i: (0, 0)),
            pl.BlockSpec((1, OUT_DIM), lambda i: (0, 0)),
        ],
        out_specs=pl.BlockSpec((NB, OUT_DIM), lambda i: (i, 0)),
        out_shape=jax.ShapeDtypeStruct((N_NODES, OUT_DIM), jnp.float32),
    )(x, agg2, batch2d, u, W2, b2r)
    return out
